# baseline (speedup 1.0000x reference)
import jax
import jax.numpy as jnp
from jax import lax
from jax.experimental import pallas as pl
from jax.experimental.pallas import tpu as pltpu
from jax.experimental.pallas import tpu_sc as plsc

_B = 16384
_D = 32
_NC, _NS = 2, 16
_NW = _NC * _NS
_BPW = _B // _NW
_NT = 8


def _sc_gather(idx, tables):
    mesh = plsc.VectorSubcoreMesh(core_axis_name="c", subcore_axis_name="s")

    half = _BPW // 2

    def body(idx_hbm, t0, t1, t2, t3, t4, t5, t6, t7, out_hbm, idx_v,
             rows4_v, rows_v, sem):
        tabs = (t0, t1, t2, t3, t4, t5, t6, t7)
        wid = lax.axis_index("s") * _NC + lax.axis_index("c")
        pltpu.sync_copy(idx_hbm.at[wid], idx_v)
        for t in range(_NT):
            for hb in range(2):
                hbase = hb * half

                def grp(g, carry, t=t, hbase=hbase):
                    base = g * 16
                    v = idx_v[t, pl.ds(hbase + base, 16)]
                    qv = lax.shift_right_logical(v, 2)
                    for k in range(16):
                        pltpu.async_copy(tabs[t].at[pl.ds(qv[k], 1)],
                                         rows4_v.at[pl.ds(base + k, 1)], sem)
                    return carry

                lax.fori_loop(0, half // 16, grp, 0)
                pltpu.make_async_copy(tabs[t].at[pl.ds(0, half)], rows4_v,
                                      sem).wait()

                def ext(g, carry, t=t, hbase=hbase):
                    base = g * 16
                    v = idx_v[t, pl.ds(hbase + base, 16)]
                    sv = lax.bitwise_and(v, 3) * 32
                    for k in range(16):
                        s = sv[k]
                        for h in (0, 16):
                            rows_v[hbase + base + k, pl.ds(h, 16)] = (
                                rows4_v[base + k, pl.ds(s + h, 16)])
                    return carry

                lax.fori_loop(0, half // 16, ext, 0)
            pltpu.sync_copy(rows_v, out_hbm.at[t, wid])

    f = pl.kernel(
        body,
        out_type=jax.ShapeDtypeStruct((_NT, _NW, _BPW, _D), jnp.float32),
        mesh=mesh,
        scratch_types=[
            pltpu.VMEM((_NT, _BPW), jnp.int32),
            pltpu.VMEM((_BPW // 2, 128), jnp.float32),
            pltpu.VMEM((_BPW, _D), jnp.float32),
            pltpu.SemaphoreType.DMA,
        ],
    )
    return f(idx, *tables)


def _mlp_body(parts_ref, feat_ref, cl_ref, w1_ref, b1_ref, w2_ref, b2_ref,
              w3_ref, b3_ref, wf_ref, bf_ref, out_ref):
    p = parts_ref[...]
    cl = cl_ref[...]
    w1 = w1_ref[...]
    gmf = p[0] * p[1]

    acc = jnp.dot(p[2], w1[0:32], preferred_element_type=jnp.float32)
    acc += jnp.dot(p[3], w1[32:64], preferred_element_type=jnp.float32)
    acc += jnp.dot(p[4], w1[64:96], preferred_element_type=jnp.float32)
    for j in range(3):
        m = (cl[:, j:j + 1] != 0).astype(jnp.float32)
        acc += jnp.dot(p[5 + j] * m, w1[96 + 32 * j:128 + 32 * j],
                       preferred_element_type=jnp.float32)
    acc += jnp.dot(feat_ref[...], w1[192:194],
                   preferred_element_type=jnp.float32)
    h = jnp.maximum(acc + b1_ref[...], 0.0)
    h = jnp.maximum(
        jnp.dot(h, w2_ref[...], preferred_element_type=jnp.float32)
        + b2_ref[...], 0.0)
    h = jnp.maximum(
        jnp.dot(h, w3_ref[...], preferred_element_type=jnp.float32)
        + b3_ref[...], 0.0)
    wf = wf_ref[...]
    z = (jnp.dot(gmf, wf[0:32], preferred_element_type=jnp.float32)
         + jnp.dot(h, wf[32:64], preferred_element_type=jnp.float32)
         + bf_ref[...])
    out_ref[...] = 1.0 / (1.0 + jnp.exp(-z))


def _tc_mlp(parts, features, cat_levels, w1, b1, w2, b2, w3, b3, wf, bf):
    bs = 2048
    return pl.pallas_call(
        _mlp_body,
        grid=(_B // bs,),
        in_specs=[
            pl.BlockSpec((_NT, bs, _D), lambda i: (0, i, 0)),
            pl.BlockSpec((bs, 2), lambda i: (i, 0)),
            pl.BlockSpec((bs, 3), lambda i: (i, 0)),
            pl.BlockSpec((194, 128), lambda i: (0, 0)),
            pl.BlockSpec((128,), lambda i: (0,)),
            pl.BlockSpec((128, 64), lambda i: (0, 0)),
            pl.BlockSpec((64,), lambda i: (0,)),
            pl.BlockSpec((64, 32), lambda i: (0, 0)),
            pl.BlockSpec((32,), lambda i: (0,)),
            pl.BlockSpec((64, 1), lambda i: (0, 0)),
            pl.BlockSpec((1,), lambda i: (0,)),
        ],
        out_specs=pl.BlockSpec((bs, 1), lambda i: (i, 0)),
        out_shape=jax.ShapeDtypeStruct((_B, 1), jnp.float32),
    )(parts, features, cat_levels, w1, b1, w2, b2, w3, b3, wf, bf)


def kernel(customer, product, seller, features, cat_levels, cust_gmf,
           prod_gmf, cust_mlp, prod_mlp, seller_mlp, cat0, cat1, cat2, W1, b1,
           W2, b2, W3, b3, Wf, bf):
    idx8 = jnp.stack([
        customer, product, customer, product, seller,
        cat_levels[:, 0], cat_levels[:, 1], cat_levels[:, 2]
    ]).astype(jnp.int32)
    idx = idx8.reshape(_NT, _NW, _BPW).transpose(1, 0, 2)
    tabs = [cust_gmf, prod_gmf, cust_mlp, prod_mlp, seller_mlp, cat0, cat1,
            cat2]
    tabs4 = [t.reshape(t.shape[0] // 4, 128) for t in tabs]
    parts = _sc_gather(idx, tabs4)
    parts = parts.reshape(_NT, _B, _D)
    out = _tc_mlp(parts, features, cat_levels.astype(jnp.int32), W1, b1, W2,
                  b2, W3, b3, Wf, bf)
    return out.reshape(_B)


# speedup vs baseline: 2.6059x; 2.6059x over previous
import jax
import jax.numpy as jnp
from jax import lax
from jax.experimental import pallas as pl
from jax.experimental.pallas import tpu as pltpu
from jax.experimental.pallas import tpu_sc as plsc

_B = 16384
_D = 32
_NC, _NS = 2, 16
_NW = _NC * _NS
_BPW = _B // _NW
_NT = 8


def _sc_gather(idx, tables):
    mesh = plsc.VectorSubcoreMesh(core_axis_name="c", subcore_axis_name="s")

    def body(idx_hbm, t0, t1, t2, t3, t4, t5, t6, t7, out_hbm, idx_v,
             rows_v, sem):
        tabs = (t0, t1, t2, t3, t4, t5, t6, t7)
        wid = lax.axis_index("s") * _NC + lax.axis_index("c")
        pltpu.sync_copy(idx_hbm.at[wid], idx_v)
        for t in range(_NT):

            def grp(g, carry, t=t):
                base = g * 16
                v = idx_v[t, pl.ds(base, 16)]
                if t == 0 or t >= 4:
                    for k in range(16):
                        pltpu.async_copy(tabs[t].at[pl.ds(v[k], 1)],
                                         rows_v.at[pl.ds(base + k, 1)], sem)
                else:
                    tiv = lax.shift_right_logical(v, 3)
                    siv = lax.bitwise_and(v, 7)
                    for k in range(16):
                        pltpu.async_copy(tabs[t].at[tiv[k], siv[k]],
                                         rows_v.at[base + k], sem)
                return carry

            lax.fori_loop(0, _BPW // 16, grp, 0)
            pltpu.make_async_copy(out_hbm.at[t, wid], rows_v, sem).wait()
            pltpu.sync_copy(rows_v, out_hbm.at[t, wid])

    f = pl.kernel(
        body,
        out_type=jax.ShapeDtypeStruct((_NT, _NW, _BPW, _D), jnp.float32),
        mesh=mesh,
        scratch_types=[
            pltpu.VMEM((_NT, _BPW), jnp.int32),
            pltpu.VMEM((_BPW, _D), jnp.float32),
            pltpu.SemaphoreType.DMA,
        ],
    )
    return f(idx, *tables)


def _mlp_body(parts_ref, feat_ref, cl_ref, w1_ref, b1_ref, w2_ref, b2_ref,
              w3_ref, b3_ref, wf_ref, bf_ref, out_ref):
    p = parts_ref[...]
    cl = cl_ref[...]
    w1 = w1_ref[...]
    gmf = p[0] * p[1]

    acc = jnp.dot(p[2], w1[0:32], preferred_element_type=jnp.float32)
    acc += jnp.dot(p[3], w1[32:64], preferred_element_type=jnp.float32)
    acc += jnp.dot(p[4], w1[64:96], preferred_element_type=jnp.float32)
    for j in range(3):
        m = (cl[:, j:j + 1] != 0).astype(jnp.float32)
        acc += jnp.dot(p[5 + j] * m, w1[96 + 32 * j:128 + 32 * j],
                       preferred_element_type=jnp.float32)
    acc += jnp.dot(feat_ref[...], w1[192:194],
                   preferred_element_type=jnp.float32)
    h = jnp.maximum(acc + b1_ref[...], 0.0)
    h = jnp.maximum(
        jnp.dot(h, w2_ref[...], preferred_element_type=jnp.float32)
        + b2_ref[...], 0.0)
    h = jnp.maximum(
        jnp.dot(h, w3_ref[...], preferred_element_type=jnp.float32)
        + b3_ref[...], 0.0)
    wf = wf_ref[...]
    z = (jnp.dot(gmf, wf[0:32], preferred_element_type=jnp.float32)
         + jnp.dot(h, wf[32:64], preferred_element_type=jnp.float32)
         + bf_ref[...])
    out_ref[...] = 1.0 / (1.0 + jnp.exp(-z))


def _tc_mlp(parts, features, cat_levels, w1, b1, w2, b2, w3, b3, wf, bf):
    bs = 2048
    return pl.pallas_call(
        _mlp_body,
        grid=(_B // bs,),
        in_specs=[
            pl.BlockSpec((_NT, bs, _D), lambda i: (0, i, 0)),
            pl.BlockSpec((bs, 2), lambda i: (i, 0)),
            pl.BlockSpec((bs, 3), lambda i: (i, 0)),
            pl.BlockSpec((194, 128), lambda i: (0, 0)),
            pl.BlockSpec((128,), lambda i: (0,)),
            pl.BlockSpec((128, 64), lambda i: (0, 0)),
            pl.BlockSpec((64,), lambda i: (0,)),
            pl.BlockSpec((64, 32), lambda i: (0, 0)),
            pl.BlockSpec((32,), lambda i: (0,)),
            pl.BlockSpec((64, 1), lambda i: (0, 0)),
            pl.BlockSpec((1,), lambda i: (0,)),
        ],
        out_specs=pl.BlockSpec((bs, 1), lambda i: (i, 0)),
        out_shape=jax.ShapeDtypeStruct((_B, 1), jnp.float32),
    )(parts, features, cat_levels, w1, b1, w2, b2, w3, b3, wf, bf)


def kernel(customer, product, seller, features, cat_levels, cust_gmf,
           prod_gmf, cust_mlp, prod_mlp, seller_mlp, cat0, cat1, cat2, W1, b1,
           W2, b2, W3, b3, Wf, bf):
    idx8 = jnp.stack([
        customer, product, customer, product, seller,
        cat_levels[:, 0], cat_levels[:, 1], cat_levels[:, 2]
    ]).astype(jnp.int32)
    idx = idx8.reshape(_NT, _NW, _BPW).transpose(1, 0, 2)
    tabs = [cust_gmf, prod_gmf, cust_mlp, prod_mlp, seller_mlp, cat0, cat1,
            cat2]
    tabs3 = [tabs[0]] + [t.reshape(t.shape[0] // 8, 8, _D)
                         for t in tabs[1:4]] + tabs[4:]
    parts = _sc_gather(idx, tabs3)
    parts = parts.reshape(_NT, _B, _D)
    out = _tc_mlp(parts, features, cat_levels.astype(jnp.int32), W1, b1, W2,
                  b2, W3, b3, Wf, bf)
    return out.reshape(_B)
